# Initial kernel scaffold; baseline (speedup 1.0000x reference)
import jax
import jax.numpy as jnp
from jax.experimental import pallas as pl


def kernel(pred, target, class_weights):
    raise NotImplementedError("write your pallas kernel here")


# speedup vs baseline: 32.4864x; 32.4864x over previous
import functools

import jax
import jax.numpy as jnp
from jax import lax
from jax.experimental import pallas as pl
from jax.experimental.pallas import tpu as pltpu
from jax.experimental.pallas import tpu_sc as plsc

NCLS = 3
NPIX = 512 * 512
NROWS = 16 * NCLS
NBINS = 4096
SCALE = 64.0
INV_SCALE = 1.0 / SCALE
NSTAT = 4
HSIZE = NCLS * NSTAT * NBINS
CHUNK = 8192
LANES = 16
NSUB = 32
HALF = NPIX // 2


def _sc_hist(pred_hbm, tgt_hbm, hist_out, g_out, hist_v, pbuf, tbuf, g_v):
    wid = lax.axis_index("s") * 2 + lax.axis_index("c")

    def zero_body(i, carry):
        hist_v[pl.ds(i * LANES, LANES)] = jnp.zeros((LANES,), jnp.float32)
        return carry

    lax.fori_loop(0, HSIZE // LANES, zero_body, 0)
    for i in range(128 // LANES):
        g_v[pl.ds(i * LANES, LANES)] = jnp.zeros((LANES,), jnp.float32)

    batch = wid // 2
    base = (wid % 2) * HALF
    ones = jnp.full((LANES,), 1.0, jnp.float32)

    for c in range(NCLS):
        row = batch * NCLS + c
        b_cnt = (c * NSTAT + 0) * NBINS
        b_pos = (c * NSTAT + 1) * NBINS
        b_s = (c * NSTAT + 2) * NBINS
        b_sp = (c * NSTAT + 3) * NBINS

        def elem_body(i, g16):
            p16 = pbuf[pl.ds(i * LANES, LANES)]
            t16 = tbuf[pl.ds(i * LANES, LANES)]
            e = 1.0 - p16 * (2.0 * t16 - 1.0)
            mask = e > 0.0
            ec = jnp.minimum(jnp.maximum(e, 0.0) * SCALE, float(NBINS - 1))
            idx = ec.astype(jnp.int32)
            lo = idx.astype(jnp.float32) * INV_SCALE
            dd = e - lo
            plsc.addupdate_scatter(hist_v, [idx + b_cnt], ones, mask=mask)
            plsc.addupdate_scatter(hist_v, [idx + b_pos], t16, mask=mask)
            plsc.addupdate_scatter(hist_v, [idx + b_s], dd, mask=mask)
            plsc.addupdate_scatter(hist_v, [idx + b_sp], dd * t16, mask=mask)
            return g16 + t16

        def chunk_body(k, g16):
            start = pl.multiple_of(base + k * CHUNK, CHUNK)
            pltpu.sync_copy(pred_hbm.at[row, pl.ds(start, CHUNK)], pbuf)
            pltpu.sync_copy(tgt_hbm.at[row, pl.ds(start, CHUNK)], tbuf)
            return lax.fori_loop(0, CHUNK // LANES, elem_body, g16)

        g16 = lax.fori_loop(0, HALF // CHUNK, chunk_body,
                            jnp.zeros((LANES,), jnp.float32))
        g_v[pl.ds(c * LANES, LANES)] = g16

    pltpu.sync_copy(hist_v, hist_out.at[wid])
    pltpu.sync_copy(g_v, g_out.at[wid])


_hist_call = pl.kernel(
    _sc_hist,
    out_type=[
        jax.ShapeDtypeStruct((NSUB, HSIZE), jnp.float32),
        jax.ShapeDtypeStruct((NSUB, 128), jnp.float32),
    ],
    mesh=plsc.VectorSubcoreMesh(core_axis_name="c", subcore_axis_name="s"),
    compiler_params=pltpu.CompilerParams(needs_layout_passes=False),
    scratch_types=[
        pltpu.VMEM((HSIZE,), jnp.float32),
        pltpu.VMEM((CHUNK,), jnp.float32),
        pltpu.VMEM((CHUNK,), jnp.float32),
        pltpu.VMEM((128,), jnp.float32),
    ],
)


def _tc_fin(hist_ref, g_ref, cw_ref, out_ref):
    hp = lax.Precision.HIGHEST
    hs = jnp.sum(hist_ref[...], axis=0)
    g2 = jnp.sum(g_ref[...], axis=0, keepdims=True)

    i128a = lax.broadcasted_iota(jnp.int32, (128, 128), 0)
    i128b = lax.broadcasted_iota(jnp.int32, (128, 128), 1)
    u_le = jnp.where(i128a <= i128b, 1.0, 0.0)
    i32a = lax.broadcasted_iota(jnp.int32, (32, 32), 0)
    i32b = lax.broadcasted_iota(jnp.int32, (32, 32), 1)
    l_lt = jnp.where(i32b < i32a, 1.0, 0.0)

    hw = jnp.float32(INV_SCALE)

    def suffix(x):
        within = jnp.dot(x, u_le, precision=hp)
        rowtot = jnp.sum(x, axis=1, keepdims=True)
        rowpre = jnp.dot(l_lt, rowtot, precision=hp)
        return jnp.sum(x) - (within + rowpre)

    total = jnp.float32(0.0)
    for c in range(NCLS):
        cnt = hs[c * NSTAT + 0]
        pos = hs[c * NSTAT + 1]
        s_i = hs[c * NSTAT + 2]
        s_ip = hs[c * NSTAT + 3]
        g_c = jnp.sum(g2[0:1, c * LANES:(c + 1) * LANES])
        k_hi = suffix(cnt)
        p_hi = suffix(pos)
        a = g_c - p_hi
        b = g_c + k_hi - p_hi
        valid = b > 0.5
        inv = 1.0 / jnp.maximum(b, 1.0)
        f0 = jnp.where(valid, 1.0 - a * inv, 0.0)
        corr = jnp.where(valid, inv * inv * (a * s_i + (b - a) * s_ip), 0.0)
        total = total + cw_ref[c] * jnp.sum(hw * f0 + corr)
    out_ref[0, 0] = total


_fin_call = pl.pallas_call(
    _tc_fin,
    out_shape=jax.ShapeDtypeStruct((1, 1), jnp.float32),
    in_specs=[
        pl.BlockSpec(memory_space=pltpu.VMEM),
        pl.BlockSpec(memory_space=pltpu.VMEM),
        pl.BlockSpec(memory_space=pltpu.SMEM),
    ],
    out_specs=pl.BlockSpec(memory_space=pltpu.SMEM),
)


def kernel(pred, target, class_weights):
    predr = pred.reshape(NROWS, NPIX)
    tgtr = target.reshape(NROWS, NPIX)
    hist, g = _hist_call(predr, tgtr)
    hist4 = hist.reshape(NSUB, NCLS * NSTAT, 32, 128)
    loss = _fin_call(hist4, g, class_weights)
    return loss.reshape(())


# speedup vs baseline: 79.8723x; 2.4586x over previous
import jax
import jax.numpy as jnp
from jax import lax
from jax.experimental import pallas as pl
from jax.experimental.pallas import tpu as pltpu
from jax.experimental.pallas import tpu_sc as plsc

NCLS = 3
NPIX = 512 * 512
NROWS = 16 * NCLS
NBINS = 4096
SCALE = 64.0
INV_SCALE = 1.0 / SCALE
NSTAT = 4
HSIZE = NCLS * NSTAT * NBINS
CHUNK = 8192
LANES = 16
NSUB = 32
HALF = NPIX // 2
NCHUNK = HALF // CHUNK
NSTEP = NCLS * NCHUNK
NPAIR = NSTEP // 2


def _sc_hist(pred_hbm, tgt_hbm, hist_out, hist_v,
             pbuf0, tbuf0, pbuf1, tbuf1, sp0, st0, sp1, st1):
    wid = lax.axis_index("s") * 2 + lax.axis_index("c")
    batch = wid // 2
    base = (wid % 2) * HALF
    ones = jnp.full((LANES,), 1.0, jnp.float32)

    def zero_body(i, carry):
        hist_v[pl.ds(i * LANES, LANES)] = jnp.zeros((LANES,), jnp.float32)
        return carry

    lax.fori_loop(0, HSIZE // LANES, zero_body, 0)

    def addr(step):
        c = step // NCHUNK
        k = step % NCHUNK
        row = batch * NCLS + c
        start = pl.multiple_of(base + k * CHUNK, CHUNK)
        return c, row, start

    def start_dma(step, pb, tb, sp, st):
        _, row, start = addr(step)
        pltpu.async_copy(pred_hbm.at[row, pl.ds(start, CHUNK)], pb, sp)
        pltpu.async_copy(tgt_hbm.at[row, pl.ds(start, CHUNK)], tb, st)

    def wait_dma(step, pb, tb, sp, st):
        _, row, start = addr(step)
        pltpu.make_async_copy(pred_hbm.at[row, pl.ds(start, CHUNK)], pb, sp).wait()
        pltpu.make_async_copy(tgt_hbm.at[row, pl.ds(start, CHUNK)], tb, st).wait()

    def process(step, pb, tb):
        c = step // NCHUNK
        c4n = c * (NSTAT * NBINS)

        @plsc.parallel_loop(0, CHUNK // LANES, unroll=4)
        def body(i):
            p16 = pb[pl.ds(i * LANES, LANES)]
            t16 = tb[pl.ds(i * LANES, LANES)]
            e = 1.0 - p16 * (2.0 * t16 - 1.0)
            ep = jnp.maximum(e, 0.0)
            ec = jnp.minimum(ep * SCALE, float(NBINS - 1))
            idx = ec.astype(jnp.int32)
            lo = idx.astype(jnp.float32) * INV_SCALE
            dd = ep - lo
            icnt = idx + t16.astype(jnp.int32) * NBINS + c4n
            plsc.addupdate_scatter(hist_v, [icnt], ones)
            plsc.addupdate_scatter(hist_v, [icnt + 2 * NBINS], dd)

    start_dma(0, pbuf0, tbuf0, sp0, st0)

    def pair_body(j, carry):
        s0 = j * 2
        start_dma(s0 + 1, pbuf1, tbuf1, sp1, st1)
        wait_dma(s0, pbuf0, tbuf0, sp0, st0)
        process(s0, pbuf0, tbuf0)

        @pl.when(j < NPAIR - 1)
        def _():
            start_dma(s0 + 2, pbuf0, tbuf0, sp0, st0)

        wait_dma(s0 + 1, pbuf1, tbuf1, sp1, st1)
        process(s0 + 1, pbuf1, tbuf1)
        return carry

    lax.fori_loop(0, NPAIR, pair_body, 0)
    pltpu.sync_copy(hist_v, hist_out.at[wid])


_hist_call = pl.kernel(
    _sc_hist,
    out_type=jax.ShapeDtypeStruct((NSUB, HSIZE), jnp.float32),
    mesh=plsc.VectorSubcoreMesh(core_axis_name="c", subcore_axis_name="s"),
    compiler_params=pltpu.CompilerParams(needs_layout_passes=False),
    scratch_types=[
        pltpu.VMEM((HSIZE,), jnp.float32),
        pltpu.VMEM((CHUNK,), jnp.float32),
        pltpu.VMEM((CHUNK,), jnp.float32),
        pltpu.VMEM((CHUNK,), jnp.float32),
        pltpu.VMEM((CHUNK,), jnp.float32),
        pltpu.SemaphoreType.DMA,
        pltpu.SemaphoreType.DMA,
        pltpu.SemaphoreType.DMA,
        pltpu.SemaphoreType.DMA,
    ],
)


def _tc_fin(hist_ref, cw_ref, out_ref):
    hp = lax.Precision.HIGHEST
    hs = jnp.sum(hist_ref[...], axis=0)

    i128a = lax.broadcasted_iota(jnp.int32, (128, 128), 0)
    i128b = lax.broadcasted_iota(jnp.int32, (128, 128), 1)
    u_le = jnp.where(i128a <= i128b, 1.0, 0.0)
    i32a = lax.broadcasted_iota(jnp.int32, (32, 32), 0)
    i32b = lax.broadcasted_iota(jnp.int32, (32, 32), 1)
    l_lt = jnp.where(i32b < i32a, 1.0, 0.0)

    def suffix(x):
        within = jnp.dot(x, u_le, precision=hp)
        rowtot = jnp.sum(x, axis=1, keepdims=True)
        rowpre = jnp.dot(l_lt, rowtot, precision=hp)
        return jnp.sum(x) - (within + rowpre)

    total = jnp.float32(0.0)
    for c in range(NCLS):
        cnt_n = hs[c * NSTAT + 0]
        cnt_p = hs[c * NSTAT + 1]
        sum_n = hs[c * NSTAT + 2]
        sum_p = hs[c * NSTAT + 3]
        cnt = cnt_n + cnt_p
        s_i = sum_n + sum_p
        g_c = jnp.sum(cnt_p)
        k_hi = suffix(cnt)
        p_hi = suffix(cnt_p)
        a = g_c - p_hi
        b = g_c + k_hi - p_hi
        valid = b > 0.5
        inv = 1.0 / jnp.maximum(b, 1.0)
        f0 = jnp.where(valid, 1.0 - a * inv, 0.0)
        corr = jnp.where(valid, inv * inv * (a * s_i + (b - a) * sum_p), 0.0)
        total = total + cw_ref[c] * jnp.sum(INV_SCALE * f0 + corr)
    out_ref[0, 0] = total


_fin_call = pl.pallas_call(
    _tc_fin,
    out_shape=jax.ShapeDtypeStruct((1, 1), jnp.float32),
    in_specs=[
        pl.BlockSpec(memory_space=pltpu.VMEM),
        pl.BlockSpec(memory_space=pltpu.SMEM),
    ],
    out_specs=pl.BlockSpec(memory_space=pltpu.SMEM),
)


def kernel(pred, target, class_weights):
    predr = pred.reshape(NROWS, NPIX)
    tgtr = target.reshape(NROWS, NPIX)
    hist = _hist_call(predr, tgtr)
    hist4 = hist.reshape(NSUB, NCLS * NSTAT, 32, 128)
    loss = _fin_call(hist4, class_weights)
    return loss.reshape(())
